# baseline (speedup 1.0000x reference)
import functools

import jax
import jax.numpy as jnp
from jax import lax
from jax.experimental import pallas as pl
from jax.experimental.pallas import tpu as pltpu
from jax.experimental.pallas import tpu_sc as plsc

N = 10000
D = 128
E = 320000
NC = 2
NS = 16
NW = NC * NS
CH = 112
NCH = 90
EPT = NCH * CH
EPAD = NW * EPT
EDS = 4 * CH
NPAD = 10240
RPT = NPAD // NS
DPT = NPAD // NS

_MESH = dict(core_axis_name="c", subcore_axis_name="s")


def _make_agg(with_deg):
    out_type = [jax.ShapeDtypeStruct((NC, NPAD, D), jnp.float32)]
    scratch = [
        pltpu.VMEM((3 * EDS,), jnp.int32),
        pltpu.VMEM((2, CH), jnp.int32),
        pltpu.VMEM((3, CH, D), jnp.float32),
        pltpu.VMEM_SHARED((NPAD, D), jnp.float32),
        pltpu.SemaphoreType.DMA,
        pltpu.SemaphoreType.DMA,
        pltpu.SemaphoreType.DMA,
    ]
    if with_deg:
        out_type.append(jax.ShapeDtypeStruct((NC, NPAD), jnp.float32))
        scratch += [
            pltpu.VMEM((2, CH), jnp.float32),
            pltpu.VMEM((DPT,), jnp.float32),
            pltpu.VMEM_SHARED((NPAD,), jnp.float32),
            pltpu.SemaphoreType.DMA,
        ]

    @functools.partial(
        pl.kernel,
        out_type=out_type if with_deg else out_type[0],
        mesh=plsc.VectorSubcoreMesh(**_MESH),
        compiler_params=pltpu.CompilerParams(needs_layout_passes=False),
        scratch_types=scratch,
    )
    def agg(h_hbm, ed_hbm, out_hbm, *rest):
        if with_deg:
            (outd_hbm, ed_v, sdst_v, rows_v, acc_sh, dwsem, gsem, ssem,
             val_v, zd_v, dacc_sh, dsem) = rest
        else:
            ed_v, sdst_v, rows_v, acc_sh, dwsem, gsem, ssem = rest
        cid = lax.axis_index("c")
        sid = lax.axis_index("s")
        wid = sid * NC + cid
        ebase = wid * (NCH * EDS)
        z16 = jnp.zeros((16,), jnp.float32)

        def zbody(r, carry):
            for j in range(D // 16):
                rows_v[0, r, pl.ds(j * 16, 16)] = z16
            return carry

        lax.fori_loop(0, CH, zbody, 0)
        base = sid * RPT
        for k in range(RPT // CH):
            pltpu.sync_copy(rows_v.at[0], acc_sh.at[pl.ds(base + k * CH, CH)])
        rem = RPT - (RPT // CH) * CH
        pltpu.sync_copy(rows_v.at[0, pl.ds(0, rem)],
                        acc_sh.at[pl.ds(base + (RPT // CH) * CH, rem)])
        if with_deg:
            def zdbody(i, carry):
                zd_v[pl.ds(i * 16, 16)] = z16
                return carry

            lax.fori_loop(0, DPT // 16, zdbody, 0)
            pltpu.sync_copy(zd_v, dacc_sh.at[pl.ds(sid * DPT, DPT)])
        plsc.subcore_barrier()

        def fire_dw(c, b3):
            pltpu.async_copy(ed_hbm.at[pl.ds(ebase + c * EDS, EDS)],
                             ed_v.at[pl.ds(b3 * EDS, EDS)], dwsem)

        def wait_dw(b3):
            pltpu.make_async_copy(ed_hbm.at[pl.ds(0, EDS)],
                                  ed_v.at[pl.ds(b3 * EDS, EDS)], dwsem).wait()

        def fire_gather(b3):
            pltpu.async_copy(h_hbm.at[ed_v.at[pl.ds(b3 * EDS, CH)]],
                             rows_v.at[b3], gsem)

        def wait_gather(b3):
            pltpu.make_async_copy(h_hbm.at[ed_v.at[pl.ds(b3 * EDS, CH)]],
                                  rows_v.at[b3], gsem).wait()

        def wait_scatter(b2):
            pltpu.make_async_copy(rows_v.at[0], acc_sh.at[sdst_v.at[b2]], ssem).wait()
            if with_deg:
                pltpu.make_async_copy(val_v.at[b2],
                                      dacc_sh.at[sdst_v.at[b2]], dsem).wait()

        for b3 in range(3):
            fire_dw(b3, b3)
        wait_dw(0)
        fire_gather(0)

        def gbody(g, carry):
            for u in range(6):
                c = 6 * g + u
                b3 = u % 3
                b2 = u % 2
                b3n = (u + 1) % 3
                if u >= 2:
                    wait_scatter(b2)
                else:
                    @pl.when(g > 0)
                    def _():
                        wait_scatter(b2)
                if u < 5:
                    wait_dw(b3n)
                    fire_gather(b3n)
                else:
                    @pl.when(g < NCH // 6 - 1)
                    def _():
                        wait_dw(b3n)
                        fire_gather(b3n)
                wait_gather(b3)
                for j in range(CH // 16):
                    sl = pl.ds(j * 16, 16)
                    sdst_v[b2, sl] = ed_v[pl.ds(b3 * EDS + CH + j * 16, 16)]
                    if with_deg:
                        vv = ed_v[pl.ds(b3 * EDS + 3 * CH + j * 16, 16)]
                        val_v[b2, sl] = plsc.bitcast(vv, jnp.float32)

                def ebody(i, ecarry):
                    for uu in range(2):
                        e = 2 * i + uu
                        w16i = plsc.load_gather(
                            ed_v.at[pl.ds(b3 * EDS + 2 * CH, CH)],
                            [jnp.full((16,), e, jnp.int32)])
                        w16 = plsc.bitcast(w16i, jnp.float32)
                        for j in range(D // 16):
                            sl = pl.ds(j * 16, 16)
                            rows_v[b3, e, sl] = rows_v[b3, e, sl] * w16
                    return ecarry

                lax.fori_loop(0, CH // 2, ebody, 0)
                pltpu.async_copy(rows_v.at[b3], acc_sh.at[sdst_v.at[b2]],
                                 ssem, add=True)
                if with_deg:
                    pltpu.async_copy(val_v.at[b2], dacc_sh.at[sdst_v.at[b2]],
                                     dsem, add=True)

                if u <= 2:
                    fire_dw(c + 3, b3)
                else:
                    @pl.when(g < NCH // 6 - 1)
                    def _():
                        fire_dw(c + 3, b3)
            return carry

        lax.fori_loop(0, NCH // 6, gbody, 0)
        wait_scatter(0)
        wait_scatter(1)
        plsc.subcore_barrier()
        pltpu.sync_copy(acc_sh.at[pl.ds(sid * RPT, RPT)],
                        out_hbm.at[cid, pl.ds(sid * RPT, RPT)])
        if with_deg:
            pltpu.sync_copy(dacc_sh.at[pl.ds(sid * DPT, DPT)],
                            outd_hbm.at[cid, pl.ds(sid * DPT, DPT)])

    return agg


_agg_kernel = _make_agg(False)
_agg_deg_kernel = _make_agg(True)


BN = 1000


def _dense_body(p_ref, iv_ref, h_ref, wr_ref, b_ref, wo_ref, o_ref):
    m = (p_ref[0] + p_ref[1]) * iv_ref[...]
    acc = lax.dot_general(m, wr_ref[...], (((1,), (1,)), ((), ())),
                          preferred_element_type=jnp.float32)
    acc = acc + lax.dot_general(h_ref[...], wo_ref[...], (((1,), (1,)), ((), ())),
                                preferred_element_type=jnp.float32)
    o_ref[...] = jnp.maximum(acc + b_ref[...], 0.0)


def _dense(parts, invd, h, w_rel, b_rel, w_root):
    return pl.pallas_call(
        _dense_body,
        grid=(N // BN,),
        in_specs=[
            pl.BlockSpec((2, BN, D), lambda i: (0, i, 0)),
            pl.BlockSpec((BN, 1), lambda i: (i, 0)),
            pl.BlockSpec((BN, D), lambda i: (i, 0)),
            pl.BlockSpec((D, D), lambda i: (0, 0)),
            pl.BlockSpec((1, D), lambda i: (0, 0)),
            pl.BlockSpec((D, D), lambda i: (0, 0)),
        ],
        out_specs=pl.BlockSpec((BN, D), lambda i: (i, 0)),
        out_shape=jax.ShapeDtypeStruct((N, D), jnp.float32),
    )(parts, invd, h, w_rel, b_rel.reshape(1, D), w_root)


def kernel(x, edge_index, edge_attr, W_rel1, b_rel1, W_root1,
           W_rel2, b_rel2, W_root2, W_rel3, b_rel3, W_root3):
    src = edge_index[0]
    dst = edge_index[1]
    pad = EPAD - E
    fill = (jnp.arange(pad, dtype=jnp.int32) * 37) % N
    src_p = jnp.concatenate([src, fill])
    dst_p = jnp.concatenate([dst, fill])
    zpad = jnp.zeros((pad,), jnp.float32)
    ea_p = jnp.concatenate([edge_attr, zpad])
    val_p = jnp.concatenate([jnp.ones((E,), jnp.float32), zpad])
    ed = jnp.stack([
        src_p.reshape(NW * NCH, CH),
        dst_p.reshape(NW * NCH, CH),
        lax.bitcast_convert_type(ea_p, jnp.int32).reshape(NW * NCH, CH),
        lax.bitcast_convert_type(val_p, jnp.int32).reshape(NW * NCH, CH),
    ], axis=1).reshape(-1)

    parts, deg2 = _agg_deg_kernel(x, ed)
    deg = deg2[0, :N] + deg2[1, :N]
    invd = (1.0 / jnp.clip(deg, 1.0, None)).reshape(N, 1)

    h = _dense(parts, invd, x, W_rel1, b_rel1, W_root1)
    for w_rel, b_rel, w_root in ((W_rel2, b_rel2, W_root2),
                                 (W_rel3, b_rel3, W_root3)):
        parts = _agg_kernel(h, ed)
        h = _dense(parts, invd, h, w_rel, b_rel, w_root)
    return h


# speedup vs baseline: 1.0074x; 1.0074x over previous
import functools

import jax
import jax.numpy as jnp
from jax import lax
from jax.experimental import pallas as pl
from jax.experimental.pallas import tpu as pltpu
from jax.experimental.pallas import tpu_sc as plsc

N = 10000
D = 128
E = 320000
NC = 2
NS = 16
NW = NC * NS
CH = 112
NCH = 90
EPT = NCH * CH
EPAD = NW * EPT

NPAD = 10240
RPT = NPAD // NS
DPT = NPAD // NS

_MESH = dict(core_axis_name="c", subcore_axis_name="s")


def _make_agg(with_deg):
    EDS = (4 if with_deg else 3) * CH
    out_type = [jax.ShapeDtypeStruct((NC, NPAD, D), jnp.float32)]
    scratch = [
        pltpu.VMEM((3 * EDS,), jnp.int32),
        pltpu.VMEM((2, CH), jnp.int32),
        pltpu.VMEM((3, CH, D), jnp.float32),
        pltpu.VMEM_SHARED((NPAD, D), jnp.float32),
        pltpu.SemaphoreType.DMA,
        pltpu.SemaphoreType.DMA,
        pltpu.SemaphoreType.DMA,
    ]
    if with_deg:
        out_type.append(jax.ShapeDtypeStruct((NC, NPAD), jnp.float32))
        scratch += [
            pltpu.VMEM((2, CH), jnp.float32),
            pltpu.VMEM((DPT,), jnp.float32),
            pltpu.VMEM_SHARED((NPAD,), jnp.float32),
            pltpu.SemaphoreType.DMA,
        ]

    @functools.partial(
        pl.kernel,
        out_type=out_type if with_deg else out_type[0],
        mesh=plsc.VectorSubcoreMesh(**_MESH),
        compiler_params=pltpu.CompilerParams(needs_layout_passes=False),
        scratch_types=scratch,
    )
    def agg(h_hbm, ed_hbm, out_hbm, *rest):
        if with_deg:
            (outd_hbm, ed_v, sdst_v, rows_v, acc_sh, dwsem, gsem, ssem,
             val_v, zd_v, dacc_sh, dsem) = rest
        else:
            ed_v, sdst_v, rows_v, acc_sh, dwsem, gsem, ssem = rest
        cid = lax.axis_index("c")
        sid = lax.axis_index("s")
        wid = sid * NC + cid
        ebase = wid * (NCH * EDS)
        z16 = jnp.zeros((16,), jnp.float32)

        def zbody(r, carry):
            for j in range(D // 16):
                rows_v[0, r, pl.ds(j * 16, 16)] = z16
            return carry

        lax.fori_loop(0, CH, zbody, 0)
        base = sid * RPT
        for k in range(RPT // CH):
            pltpu.sync_copy(rows_v.at[0], acc_sh.at[pl.ds(base + k * CH, CH)])
        rem = RPT - (RPT // CH) * CH
        pltpu.sync_copy(rows_v.at[0, pl.ds(0, rem)],
                        acc_sh.at[pl.ds(base + (RPT // CH) * CH, rem)])
        if with_deg:
            def zdbody(i, carry):
                zd_v[pl.ds(i * 16, 16)] = z16
                return carry

            lax.fori_loop(0, DPT // 16, zdbody, 0)
            pltpu.sync_copy(zd_v, dacc_sh.at[pl.ds(sid * DPT, DPT)])
        plsc.subcore_barrier()

        def fire_dw(c, b3):
            pltpu.async_copy(ed_hbm.at[pl.ds(ebase + c * EDS, EDS)],
                             ed_v.at[pl.ds(b3 * EDS, EDS)], dwsem)

        def wait_dw(b3):
            pltpu.make_async_copy(ed_hbm.at[pl.ds(0, EDS)],
                                  ed_v.at[pl.ds(b3 * EDS, EDS)], dwsem).wait()

        def fire_gather(b3):
            pltpu.async_copy(h_hbm.at[ed_v.at[pl.ds(b3 * EDS, CH)]],
                             rows_v.at[b3], gsem)

        def wait_gather(b3):
            pltpu.make_async_copy(h_hbm.at[ed_v.at[pl.ds(b3 * EDS, CH)]],
                                  rows_v.at[b3], gsem).wait()

        def wait_scatter(b2):
            pltpu.make_async_copy(rows_v.at[0], acc_sh.at[sdst_v.at[b2]], ssem).wait()
            if with_deg:
                pltpu.make_async_copy(val_v.at[b2],
                                      dacc_sh.at[sdst_v.at[b2]], dsem).wait()

        for b3 in range(3):
            fire_dw(b3, b3)
        wait_dw(0)
        fire_gather(0)

        def gbody(g, carry):
            for u in range(6):
                c = 6 * g + u
                b3 = u % 3
                b2 = u % 2
                b3n = (u + 1) % 3
                if u >= 2:
                    wait_scatter(b2)
                else:
                    @pl.when(g > 0)
                    def _():
                        wait_scatter(b2)
                if u < 5:
                    wait_dw(b3n)
                    fire_gather(b3n)
                else:
                    @pl.when(g < NCH // 6 - 1)
                    def _():
                        wait_dw(b3n)
                        fire_gather(b3n)
                wait_gather(b3)
                for j in range(CH // 16):
                    sl = pl.ds(j * 16, 16)
                    sdst_v[b2, sl] = ed_v[pl.ds(b3 * EDS + CH + j * 16, 16)]
                    if with_deg:
                        vv = ed_v[pl.ds(b3 * EDS + 3 * CH + j * 16, 16)]
                        val_v[b2, sl] = plsc.bitcast(vv, jnp.float32)

                def ebody(i, ecarry):
                    for uu in range(2):
                        e = 2 * i + uu
                        w16i = plsc.load_gather(
                            ed_v.at[pl.ds(b3 * EDS + 2 * CH, CH)],
                            [jnp.full((16,), e, jnp.int32)])
                        w16 = plsc.bitcast(w16i, jnp.float32)
                        for j in range(D // 16):
                            sl = pl.ds(j * 16, 16)
                            rows_v[b3, e, sl] = rows_v[b3, e, sl] * w16
                    return ecarry

                lax.fori_loop(0, CH // 2, ebody, 0)
                pltpu.async_copy(rows_v.at[b3], acc_sh.at[sdst_v.at[b2]],
                                 ssem, add=True)
                if with_deg:
                    pltpu.async_copy(val_v.at[b2], dacc_sh.at[sdst_v.at[b2]],
                                     dsem, add=True)

                if u <= 2:
                    fire_dw(c + 3, b3)
                else:
                    @pl.when(g < NCH // 6 - 1)
                    def _():
                        fire_dw(c + 3, b3)
            return carry

        lax.fori_loop(0, NCH // 6, gbody, 0)
        wait_scatter(0)
        wait_scatter(1)
        plsc.subcore_barrier()
        pltpu.sync_copy(acc_sh.at[pl.ds(sid * RPT, RPT)],
                        out_hbm.at[cid, pl.ds(sid * RPT, RPT)])
        if with_deg:
            pltpu.sync_copy(dacc_sh.at[pl.ds(sid * DPT, DPT)],
                            outd_hbm.at[cid, pl.ds(sid * DPT, DPT)])

    return agg


_agg_kernel = _make_agg(False)
_agg_deg_kernel = _make_agg(True)


BN = 1000


def _dense_body(p_ref, iv_ref, h_ref, wr_ref, b_ref, wo_ref, o_ref):
    m = (p_ref[0] + p_ref[1]) * iv_ref[...]
    acc = lax.dot_general(m, wr_ref[...], (((1,), (1,)), ((), ())),
                          preferred_element_type=jnp.float32)
    acc = acc + lax.dot_general(h_ref[...], wo_ref[...], (((1,), (1,)), ((), ())),
                                preferred_element_type=jnp.float32)
    o_ref[...] = jnp.maximum(acc + b_ref[...], 0.0)


def _dense(parts, invd, h, w_rel, b_rel, w_root):
    return pl.pallas_call(
        _dense_body,
        grid=(N // BN,),
        in_specs=[
            pl.BlockSpec((2, BN, D), lambda i: (0, i, 0)),
            pl.BlockSpec((BN, 1), lambda i: (i, 0)),
            pl.BlockSpec((BN, D), lambda i: (i, 0)),
            pl.BlockSpec((D, D), lambda i: (0, 0)),
            pl.BlockSpec((1, D), lambda i: (0, 0)),
            pl.BlockSpec((D, D), lambda i: (0, 0)),
        ],
        out_specs=pl.BlockSpec((BN, D), lambda i: (i, 0)),
        out_shape=jax.ShapeDtypeStruct((N, D), jnp.float32),
    )(parts, invd, h, w_rel, b_rel.reshape(1, D), w_root)


def kernel(x, edge_index, edge_attr, W_rel1, b_rel1, W_root1,
           W_rel2, b_rel2, W_root2, W_rel3, b_rel3, W_root3):
    src = edge_index[0]
    dst = edge_index[1]
    pad = EPAD - E
    fill = (jnp.arange(pad, dtype=jnp.int32) * 37) % N
    src_p = jnp.concatenate([src, fill])
    dst_p = jnp.concatenate([dst, fill])
    zpad = jnp.zeros((pad,), jnp.float32)
    ea_p = jnp.concatenate([edge_attr, zpad])
    val_p = jnp.concatenate([jnp.ones((E,), jnp.float32), zpad])
    cols = [
        src_p.reshape(NW * NCH, CH),
        dst_p.reshape(NW * NCH, CH),
        lax.bitcast_convert_type(ea_p, jnp.int32).reshape(NW * NCH, CH),
    ]
    ed = jnp.stack(cols, axis=1).reshape(-1)
    ed4 = jnp.stack(
        cols + [lax.bitcast_convert_type(val_p, jnp.int32).reshape(NW * NCH, CH)],
        axis=1).reshape(-1)

    parts, deg2 = _agg_deg_kernel(x, ed4)
    deg = deg2[0, :N] + deg2[1, :N]
    invd = (1.0 / jnp.clip(deg, 1.0, None)).reshape(N, 1)

    h = _dense(parts, invd, x, W_rel1, b_rel1, W_root1)
    for w_rel, b_rel, w_root in ((W_rel2, b_rel2, W_root2),
                                 (W_rel3, b_rel3, W_root3)):
        parts = _agg_kernel(h, ed)
        h = _dense(parts, invd, h, w_rel, b_rel, w_root)
    return h


# speedup vs baseline: 1.0179x; 1.0104x over previous
import functools

import jax
import jax.numpy as jnp
from jax import lax
from jax.experimental import pallas as pl
from jax.experimental.pallas import tpu as pltpu
from jax.experimental.pallas import tpu_sc as plsc

N = 10000
D = 128
E = 320000
NC = 2
NS = 16
NW = NC * NS
CH = 112
NCH = 90
EPT = NCH * CH
EPAD = NW * EPT
NPAD = 10240
RPT = NPAD // NS
DPT = NPAD // NS

_MESH = dict(core_axis_name="c", subcore_axis_name="s")


@functools.partial(
    pl.kernel,
    out_type=jax.ShapeDtypeStruct((NC, NPAD), jnp.float32),
    mesh=plsc.VectorSubcoreMesh(**_MESH),
    compiler_params=pltpu.CompilerParams(needs_layout_passes=False),
    scratch_types=[
        pltpu.VMEM((3, CH), jnp.int32),
        pltpu.VMEM((3, CH), jnp.float32),
        pltpu.VMEM((DPT,), jnp.float32),
        pltpu.VMEM_SHARED((NPAD,), jnp.float32),
        pltpu.SemaphoreType.DMA,
    ],
)
def _deg_kernel(dst_hbm, val_hbm, out_hbm, dst_v, val_v, zer_v, acc_sh, dwsem):
    cid = lax.axis_index("c")
    sid = lax.axis_index("s")
    wid = sid * NC + cid
    ebase = wid * EPT
    z16 = jnp.zeros((16,), jnp.float32)

    def zbody(i, carry):
        zer_v[pl.ds(i * 16, 16)] = z16
        return carry

    lax.fori_loop(0, DPT // 16, zbody, 0)
    pltpu.sync_copy(zer_v, acc_sh.at[pl.ds(sid * DPT, DPT)])
    plsc.subcore_barrier()

    def fire_dw(c, b):
        pltpu.async_copy(dst_hbm.at[pl.ds(ebase + c * CH, CH)], dst_v.at[b], dwsem)
        pltpu.async_copy(val_hbm.at[pl.ds(ebase + c * CH, CH)], val_v.at[b], dwsem)

    def wait_dw(b):
        pltpu.make_async_copy(dst_hbm.at[pl.ds(0, CH)], dst_v.at[b], dwsem).wait()
        pltpu.make_async_copy(val_hbm.at[pl.ds(0, CH)], val_v.at[b], dwsem).wait()

    for b in range(3):
        fire_dw(b, b)

    def gbody(g, carry):
        for b in range(3):
            c = 3 * g + b
            wait_dw(b)
            pltpu.sync_copy(val_v.at[b], acc_sh.at[dst_v.at[b]], add=True)

            @pl.when(g < NCH // 3 - 1)
            def _():
                fire_dw(c + 3, b)
        return carry

    lax.fori_loop(0, NCH // 3, gbody, 0)
    plsc.subcore_barrier()
    pltpu.sync_copy(acc_sh.at[pl.ds(sid * DPT, DPT)],
                    out_hbm.at[cid, pl.ds(sid * DPT, DPT)])


@functools.partial(
    pl.kernel,
    out_type=jax.ShapeDtypeStruct((NC, NPAD, D), jnp.float32),
    mesh=plsc.VectorSubcoreMesh(**_MESH),
    compiler_params=pltpu.CompilerParams(needs_layout_passes=False),
    scratch_types=[
        pltpu.VMEM((3 * 3 * CH,), jnp.int32),
        pltpu.VMEM((3, CH), jnp.int32),
        pltpu.VMEM((3, CH, D), jnp.float32),
        pltpu.VMEM_SHARED((NPAD, D), jnp.float32),
        pltpu.SemaphoreType.DMA,
        pltpu.SemaphoreType.DMA,
        pltpu.SemaphoreType.DMA,
    ],
)
def _agg_kernel(h_hbm, ed_hbm, out_hbm,
                ed_v, sdst_v, rows_v, acc_sh, dwsem, gsem, ssem):
    cid = lax.axis_index("c")
    sid = lax.axis_index("s")
    wid = sid * NC + cid
    ebase = wid * (NCH * 3 * CH)
    z16 = jnp.zeros((16,), jnp.float32)

    def zbody(r, carry):
        for j in range(D // 16):
            rows_v[0, r, pl.ds(j * 16, 16)] = z16
        return carry

    lax.fori_loop(0, CH, zbody, 0)
    base = sid * RPT
    for k in range(RPT // CH):
        pltpu.sync_copy(rows_v.at[0], acc_sh.at[pl.ds(base + k * CH, CH)])
    rem = RPT - (RPT // CH) * CH
    pltpu.sync_copy(rows_v.at[0, pl.ds(0, rem)],
                    acc_sh.at[pl.ds(base + (RPT // CH) * CH, rem)])
    plsc.subcore_barrier()

    def fire_dw(c, b):
        pltpu.async_copy(ed_hbm.at[pl.ds(ebase + c * (3 * CH), 3 * CH)],
                         ed_v.at[pl.ds(b * 3 * CH, 3 * CH)], dwsem)

    def wait_dw(b):
        pltpu.make_async_copy(ed_hbm.at[pl.ds(0, 3 * CH)],
                              ed_v.at[pl.ds(b * 3 * CH, 3 * CH)], dwsem).wait()

    def fire_gather(b):
        pltpu.async_copy(h_hbm.at[ed_v.at[pl.ds(b * 3 * CH, CH)]], rows_v.at[b], gsem)

    def wait_gather(b):
        pltpu.make_async_copy(h_hbm.at[ed_v.at[pl.ds(b * 3 * CH, CH)]], rows_v.at[b], gsem).wait()

    def wait_scatter(b):
        pltpu.make_async_copy(rows_v.at[b], acc_sh.at[sdst_v.at[b]], ssem).wait()

    for b in range(3):
        fire_dw(b, b)
    wait_dw(0)
    fire_gather(0)

    def gbody(g, carry):
        for b in range(3):
            c = 3 * g + b
            bn = (b + 1) % 3
            if b == 2:
                wait_scatter(bn)
            else:
                @pl.when(g > 0)
                def _():
                    wait_scatter(bn)
            if b == 2:
                @pl.when(g < NCH // 3 - 1)
                def _():
                    wait_dw(bn)
                    fire_gather(bn)
            else:
                wait_dw(bn)
                fire_gather(bn)
            wait_gather(b)
            for j in range(CH // 16):
                sl = pl.ds(j * 16, 16)
                sdst_v[b, sl] = ed_v[pl.ds(b * 3 * CH + CH + j * 16, 16)]

            def ebody(i, ecarry):
                for u in range(4):
                    e = 4 * i + u
                    w16i = plsc.load_gather(ed_v.at[pl.ds(b * 3 * CH + 2 * CH, CH)], [jnp.full((16,), e, jnp.int32)])
                    w16 = plsc.bitcast(w16i, jnp.float32)
                    for j in range(D // 16):
                        sl = pl.ds(j * 16, 16)
                        rows_v[b, e, sl] = rows_v[b, e, sl] * w16
                return ecarry

            lax.fori_loop(0, CH // 4, ebody, 0)
            pltpu.async_copy(rows_v.at[b], acc_sh.at[sdst_v.at[b]], ssem, add=True)

            @pl.when(g < NCH // 3 - 1)
            def _():
                fire_dw(c + 3, b)
        return carry

    lax.fori_loop(0, NCH // 3, gbody, 0)
    wait_scatter(1)
    wait_scatter(2)
    plsc.subcore_barrier()
    pltpu.sync_copy(acc_sh.at[pl.ds(sid * RPT, RPT)],
                    out_hbm.at[cid, pl.ds(sid * RPT, RPT)])


BN = 1000


def _dense_body(p_ref, iv_ref, h_ref, wr_ref, b_ref, wo_ref, o_ref):
    m = (p_ref[0] + p_ref[1]) * iv_ref[...]
    acc = lax.dot_general(m, wr_ref[...], (((1,), (1,)), ((), ())),
                          preferred_element_type=jnp.float32)
    acc = acc + lax.dot_general(h_ref[...], wo_ref[...], (((1,), (1,)), ((), ())),
                                preferred_element_type=jnp.float32)
    o_ref[...] = jnp.maximum(acc + b_ref[...], 0.0)


def _dense(parts, invd, h, w_rel, b_rel, w_root):
    return pl.pallas_call(
        _dense_body,
        grid=(N // BN,),
        in_specs=[
            pl.BlockSpec((2, BN, D), lambda i: (0, i, 0)),
            pl.BlockSpec((BN, 1), lambda i: (i, 0)),
            pl.BlockSpec((BN, D), lambda i: (i, 0)),
            pl.BlockSpec((D, D), lambda i: (0, 0)),
            pl.BlockSpec((1, D), lambda i: (0, 0)),
            pl.BlockSpec((D, D), lambda i: (0, 0)),
        ],
        out_specs=pl.BlockSpec((BN, D), lambda i: (i, 0)),
        out_shape=jax.ShapeDtypeStruct((N, D), jnp.float32),
    )(parts, invd, h, w_rel, b_rel.reshape(1, D), w_root)


def kernel(x, edge_index, edge_attr, W_rel1, b_rel1, W_root1,
           W_rel2, b_rel2, W_root2, W_rel3, b_rel3, W_root3):
    src = edge_index[0]
    dst = edge_index[1]
    pad = EPAD - E
    fill = (jnp.arange(pad, dtype=jnp.int32) * 37) % N
    src_p = jnp.concatenate([src, fill])
    dst_p = jnp.concatenate([dst, fill])
    zpad = jnp.zeros((pad,), jnp.float32)
    ea_p = jnp.concatenate([edge_attr, zpad])
    val_p = jnp.concatenate([jnp.ones((E,), jnp.float32), zpad])
    ed = jnp.stack([
        src_p.reshape(NW * NCH, CH),
        dst_p.reshape(NW * NCH, CH),
        lax.bitcast_convert_type(ea_p, jnp.int32).reshape(NW * NCH, CH),
    ], axis=1).reshape(-1)

    deg2 = _deg_kernel(dst_p, val_p)
    deg = deg2[0, :N] + deg2[1, :N]
    invd = (1.0 / jnp.clip(deg, 1.0, None)).reshape(N, 1)

    h = x
    for w_rel, b_rel, w_root in ((W_rel1, b_rel1, W_root1),
                                 (W_rel2, b_rel2, W_root2),
                                 (W_rel3, b_rel3, W_root3)):
        parts = _agg_kernel(h, ed)
        h = _dense(parts, invd, h, w_rel, b_rel, w_root)
    return h


# speedup vs baseline: 1.0316x; 1.0135x over previous
import functools

import jax
import jax.numpy as jnp
from jax import lax
from jax.experimental import pallas as pl
from jax.experimental.pallas import tpu as pltpu
from jax.experimental.pallas import tpu_sc as plsc

N = 10000
D = 128
E = 320000
NC = 2
NS = 16
NW = NC * NS
CH = 112
NCH = 90
EPT = NCH * CH
EPAD = NW * EPT
NPAD = 10240
RPT = NPAD // NS
DPT = NPAD // NS

_MESH = dict(core_axis_name="c", subcore_axis_name="s")


@functools.partial(
    pl.kernel,
    out_type=jax.ShapeDtypeStruct((NC, NPAD), jnp.float32),
    mesh=plsc.VectorSubcoreMesh(**_MESH),
    compiler_params=pltpu.CompilerParams(needs_layout_passes=False),
    scratch_types=[
        pltpu.VMEM((3, CH), jnp.int32),
        pltpu.VMEM((3, CH), jnp.float32),
        pltpu.VMEM((DPT,), jnp.float32),
        pltpu.VMEM_SHARED((NPAD,), jnp.float32),
        pltpu.SemaphoreType.DMA,
    ],
)
def _deg_kernel(dst_hbm, val_hbm, out_hbm, dst_v, val_v, zer_v, acc_sh, dwsem):
    cid = lax.axis_index("c")
    sid = lax.axis_index("s")
    wid = sid * NC + cid
    ebase = wid * EPT
    z16 = jnp.zeros((16,), jnp.float32)

    def zbody(i, carry):
        zer_v[pl.ds(i * 16, 16)] = z16
        return carry

    lax.fori_loop(0, DPT // 16, zbody, 0)
    pltpu.sync_copy(zer_v, acc_sh.at[pl.ds(sid * DPT, DPT)])
    plsc.subcore_barrier()

    def fire_dw(c, b):
        pltpu.async_copy(dst_hbm.at[pl.ds(ebase + c * CH, CH)], dst_v.at[b], dwsem)
        pltpu.async_copy(val_hbm.at[pl.ds(ebase + c * CH, CH)], val_v.at[b], dwsem)

    def wait_dw(b):
        pltpu.make_async_copy(dst_hbm.at[pl.ds(0, CH)], dst_v.at[b], dwsem).wait()
        pltpu.make_async_copy(val_hbm.at[pl.ds(0, CH)], val_v.at[b], dwsem).wait()

    for b in range(3):
        fire_dw(b, b)

    def gbody(g, carry):
        for b in range(3):
            c = 3 * g + b
            wait_dw(b)
            pltpu.sync_copy(val_v.at[b], acc_sh.at[dst_v.at[b]], add=True)

            @pl.when(g < NCH // 3 - 1)
            def _():
                fire_dw(c + 3, b)
        return carry

    lax.fori_loop(0, NCH // 3, gbody, 0)
    plsc.subcore_barrier()
    pltpu.sync_copy(acc_sh.at[pl.ds(sid * DPT, DPT)],
                    out_hbm.at[cid, pl.ds(sid * DPT, DPT)])


@functools.partial(
    pl.kernel,
    out_type=jax.ShapeDtypeStruct((NC, NPAD, D), jnp.float32),
    mesh=plsc.VectorSubcoreMesh(**_MESH),
    compiler_params=pltpu.CompilerParams(needs_layout_passes=False),
    scratch_types=[
        pltpu.VMEM((3 * 3 * CH,), jnp.int32),
        pltpu.VMEM((3, CH), jnp.int32),
        pltpu.VMEM((3, CH, D), jnp.float32),
        pltpu.VMEM_SHARED((NPAD, D), jnp.float32),
        pltpu.SemaphoreType.DMA,
        pltpu.SemaphoreType.DMA,
        pltpu.SemaphoreType.DMA,
    ],
)
def _agg_kernel(h_hbm, ed_hbm, out_hbm,
                ed_v, sdst_v, rows_v, acc_sh, dwsem, gsem, ssem):
    cid = lax.axis_index("c")
    sid = lax.axis_index("s")
    wid = sid * NC + cid
    ebase = wid * (NCH * 3 * CH)
    z16 = jnp.zeros((16,), jnp.float32)

    def zbody(r, carry):
        for j in range(D // 16):
            rows_v[2, r, pl.ds(j * 16, 16)] = z16
        return carry

    def fire_dw(c, b):
        pltpu.async_copy(ed_hbm.at[pl.ds(ebase + c * (3 * CH), 3 * CH)],
                         ed_v.at[pl.ds(b * 3 * CH, 3 * CH)], dwsem)

    def wait_dw(b):
        pltpu.make_async_copy(ed_hbm.at[pl.ds(0, 3 * CH)],
                              ed_v.at[pl.ds(b * 3 * CH, 3 * CH)], dwsem).wait()

    def fire_gather(b):
        pltpu.async_copy(h_hbm.at[ed_v.at[pl.ds(b * 3 * CH, CH)]], rows_v.at[b], gsem)

    def wait_gather(b):
        pltpu.make_async_copy(h_hbm.at[ed_v.at[pl.ds(b * 3 * CH, CH)]], rows_v.at[b], gsem).wait()

    def wait_scatter(b):
        pltpu.make_async_copy(rows_v.at[b], acc_sh.at[sdst_v.at[b]], ssem).wait()

    for b in range(3):
        fire_dw(b, b)
    wait_dw(0)
    fire_gather(0)
    lax.fori_loop(0, CH, zbody, 0)
    base = sid * RPT
    for k in range(RPT // CH):
        pltpu.sync_copy(rows_v.at[2], acc_sh.at[pl.ds(base + k * CH, CH)])
    rem = RPT - (RPT // CH) * CH
    pltpu.sync_copy(rows_v.at[2, pl.ds(0, rem)],
                    acc_sh.at[pl.ds(base + (RPT // CH) * CH, rem)])
    plsc.subcore_barrier()

    def gbody(g, carry):
        for b in range(3):
            c = 3 * g + b
            bn = (b + 1) % 3
            if b == 2:
                wait_scatter(bn)
            else:
                @pl.when(g > 0)
                def _():
                    wait_scatter(bn)
            if b == 2:
                @pl.when(g < NCH // 3 - 1)
                def _():
                    wait_dw(bn)
                    fire_gather(bn)
            else:
                wait_dw(bn)
                fire_gather(bn)
            wait_gather(b)
            for j in range(CH // 16):
                sl = pl.ds(j * 16, 16)
                sdst_v[b, sl] = ed_v[pl.ds(b * 3 * CH + CH + j * 16, 16)]

            def ebody(i, ecarry):
                for u in range(2):
                    e = 2 * i + u
                    w16i = plsc.load_gather(ed_v.at[pl.ds(b * 3 * CH + 2 * CH, CH)], [jnp.full((16,), e, jnp.int32)])
                    w16 = plsc.bitcast(w16i, jnp.float32)
                    for j in range(D // 16):
                        sl = pl.ds(j * 16, 16)
                        rows_v[b, e, sl] = rows_v[b, e, sl] * w16
                return ecarry

            lax.fori_loop(0, CH // 2, ebody, 0)
            pltpu.async_copy(rows_v.at[b], acc_sh.at[sdst_v.at[b]], ssem, add=True)

            @pl.when(g < NCH // 3 - 1)
            def _():
                fire_dw(c + 3, b)
        return carry

    lax.fori_loop(0, NCH // 3, gbody, 0)
    wait_scatter(1)
    wait_scatter(2)
    plsc.subcore_barrier()
    pltpu.sync_copy(acc_sh.at[pl.ds(sid * RPT, RPT)],
                    out_hbm.at[cid, pl.ds(sid * RPT, RPT)])


BN = 1000


def _dense_body(p_ref, iv_ref, h_ref, wr_ref, b_ref, wo_ref, o_ref):
    m = (p_ref[0] + p_ref[1]) * iv_ref[...]
    acc = lax.dot_general(m, wr_ref[...], (((1,), (1,)), ((), ())),
                          preferred_element_type=jnp.float32)
    acc = acc + lax.dot_general(h_ref[...], wo_ref[...], (((1,), (1,)), ((), ())),
                                preferred_element_type=jnp.float32)
    o_ref[...] = jnp.maximum(acc + b_ref[...], 0.0)


def _dense(parts, invd, h, w_rel, b_rel, w_root):
    return pl.pallas_call(
        _dense_body,
        grid=(N // BN,),
        in_specs=[
            pl.BlockSpec((2, BN, D), lambda i: (0, i, 0)),
            pl.BlockSpec((BN, 1), lambda i: (i, 0)),
            pl.BlockSpec((BN, D), lambda i: (i, 0)),
            pl.BlockSpec((D, D), lambda i: (0, 0)),
            pl.BlockSpec((1, D), lambda i: (0, 0)),
            pl.BlockSpec((D, D), lambda i: (0, 0)),
        ],
        out_specs=pl.BlockSpec((BN, D), lambda i: (i, 0)),
        out_shape=jax.ShapeDtypeStruct((N, D), jnp.float32),
    )(parts, invd, h, w_rel, b_rel.reshape(1, D), w_root)


def kernel(x, edge_index, edge_attr, W_rel1, b_rel1, W_root1,
           W_rel2, b_rel2, W_root2, W_rel3, b_rel3, W_root3):
    src = edge_index[0]
    dst = edge_index[1]
    pad = EPAD - E
    fill = (jnp.arange(pad, dtype=jnp.int32) * 37) % N
    src_p = jnp.concatenate([src, fill])
    dst_p = jnp.concatenate([dst, fill])
    zpad = jnp.zeros((pad,), jnp.float32)
    ea_p = jnp.concatenate([edge_attr, zpad])
    val_p = jnp.concatenate([jnp.ones((E,), jnp.float32), zpad])
    ed = jnp.stack([
        src_p.reshape(NW * NCH, CH),
        dst_p.reshape(NW * NCH, CH),
        lax.bitcast_convert_type(ea_p, jnp.int32).reshape(NW * NCH, CH),
    ], axis=1).reshape(-1)

    deg2 = _deg_kernel(dst_p, val_p)
    deg = deg2[0, :N] + deg2[1, :N]
    invd = (1.0 / jnp.clip(deg, 1.0, None)).reshape(N, 1)

    h = x
    for w_rel, b_rel, w_root in ((W_rel1, b_rel1, W_root1),
                                 (W_rel2, b_rel2, W_root2),
                                 (W_rel3, b_rel3, W_root3)):
        parts = _agg_kernel(h, ed)
        h = _dense(parts, invd, h, w_rel, b_rel, w_root)
    return h
